# baseline (speedup 1.0000x reference)
import jax
import jax.numpy as jnp
from jax.experimental import pallas as pl
from jax.experimental.pallas import tpu as pltpu

_BL = 512
_SB = 16


def _body(tok_ref, en_ref, w_ref, eb_ref, pe_ref, out_ref):
    j = pl.program_id(0)
    k = pl.program_id(1)
    e = jax.lax.dot_general(
        w_ref[k], en_ref[:, pl.ds(j * _BL, _BL)], (((0,), (0,)), ((), ())),
        preferred_element_type=jnp.float32)
    out_ref[0, :, :] = e + eb_ref[pl.ds(k * _SB, _SB), :]
    out_ref[1:, :, :] = tok_ref[:] + pe_ref[:, pl.ds(k * _SB, _SB), :]


def kernel(tokens, energies, W, b, emb):
    batch, n_in, tsz = tokens.shape
    n_tok = emb.shape[0]
    tokens_t = tokens.transpose(1, 2, 0)
    energies_t = energies.T
    pe = emb[1:].reshape(n_in, tsz, 1)
    eb = (b + emb[0]).reshape(tsz, 1)
    w_r = W.reshape(tsz, tsz // _SB, _SB).transpose(1, 0, 2)

    grid = (batch // _BL, tsz // _SB)
    resident = pl.BlockSpec(memory_space=pltpu.VMEM)
    out_t = pl.pallas_call(
        _body,
        grid=grid,
        in_specs=[
            pl.BlockSpec((n_in, _SB, _BL), lambda j, k: (0, k, j)),
            resident,
            resident,
            resident,
            resident,
        ],
        out_specs=pl.BlockSpec((n_tok, _SB, _BL), lambda j, k: (0, k, j)),
        out_shape=jax.ShapeDtypeStruct((n_tok, tsz, batch), jnp.float32),
    )(tokens_t, energies_t, w_r, eb, pe)
    return out_t.transpose(2, 0, 1)


# speedup vs baseline: 1.0143x; 1.0143x over previous
import jax
import jax.numpy as jnp
from jax.experimental import pallas as pl
from jax.experimental.pallas import tpu as pltpu

_NBUF = 8


def _body(tok_ref, en_ref, w_ref, eb_ref, pe_ref, out_ref,
          in_buf, out_buf, e_buf, in_sems, out_sems, e_sem):
    n_in = tok_ref.shape[0]
    tsz, batch = tok_ref.shape[1], tok_ref.shape[2]

    def in_copy(i, slot):
        return pltpu.make_async_copy(tok_ref.at[i], in_buf.at[slot],
                                     in_sems.at[slot])

    def out_copy(i, slot):
        return pltpu.make_async_copy(out_buf.at[slot], out_ref.at[i + 1],
                                     out_sems.at[slot])

    for s in range(_NBUF):
        in_copy(s, s).start()
    qtr = batch // 4
    for j in range(4):
        sl = pl.ds(j * qtr, qtr)
        e_buf[:, sl] = jax.lax.dot_general(
            w_ref[:], en_ref[:, sl], (((0,), (0,)), ((), ())),
            preferred_element_type=jnp.float32) + eb_ref[:]
    pltpu.make_async_copy(e_buf, out_ref.at[0], e_sem).start()

    def chunk(i, slot, first_round):
        in_copy(i, slot).wait()
        if not first_round:
            out_copy(i - _NBUF, slot).wait()
        out_buf[slot] = in_buf[slot] + pe_ref[i]
        out_copy(i, slot).start()

        @pl.when(i + _NBUF < n_in)
        def _():
            in_copy(i + _NBUF, slot).start()

    for s in range(_NBUF):
        chunk(s, s, True)

    n_main = (n_in - _NBUF) // _NBUF

    def round_body(r, _):
        base = (r + 1) * _NBUF
        for s in range(_NBUF):
            chunk(base + s, s, False)
        return _

    jax.lax.fori_loop(0, n_main, round_body, 0)

    for i in range((n_main + 1) * _NBUF, n_in):
        chunk(i, i % _NBUF, False)

    for i in range(n_in - _NBUF, n_in):
        out_copy(i, i % _NBUF).wait()
    pltpu.make_async_copy(e_buf, out_ref.at[0], e_sem).wait()


def kernel(tokens, energies, W, b, emb):
    batch, n_in, tsz = tokens.shape
    n_tok = emb.shape[0]
    tokens_t = tokens.transpose(1, 2, 0)
    energies_t = energies.T
    pe = emb[1:].reshape(n_in, tsz, 1)
    eb = (b + emb[0]).reshape(tsz, 1)

    resident = pl.BlockSpec(memory_space=pltpu.MemorySpace.VMEM)
    hbm = pl.BlockSpec(memory_space=pl.ANY)
    out_t = pl.pallas_call(
        _body,
        in_specs=[hbm, resident, resident, resident, resident],
        out_specs=hbm,
        out_shape=jax.ShapeDtypeStruct((n_tok, tsz, batch), jnp.float32),
        scratch_shapes=[
            pltpu.VMEM((_NBUF, tsz, batch), jnp.float32),
            pltpu.VMEM((_NBUF, tsz, batch), jnp.float32),
            pltpu.VMEM((tsz, batch), jnp.float32),
            pltpu.SemaphoreType.DMA((_NBUF,)),
            pltpu.SemaphoreType.DMA((_NBUF,)),
            pltpu.SemaphoreType.DMA,
        ],
    )(tokens_t, energies_t, W, eb, pe)
    return out_t.transpose(2, 0, 1)
